# Initial kernel scaffold; baseline (speedup 1.0000x reference)
import jax
import jax.numpy as jnp
from jax.experimental import pallas as pl


def kernel(logits):
    raise NotImplementedError("write your pallas kernel here")


# speedup vs baseline: 66.3881x; 66.3881x over previous
import functools

import jax
import jax.numpy as jnp
from jax import lax
from jax.experimental import pallas as pl
from jax.experimental.pallas import tpu as pltpu
from jax.experimental.pallas import tpu_sc as plsc

R = 128
N = 32768
L = 16
NW = 32
ROWS_PER_W = R // NW
ACC = 8
STEPS = N // (L * ACC)


def _bcast_max(x):
    fwd = plsc.cummax(x)
    bwd = lax.rev(plsc.cummax(lax.rev(x, (0,))), (0,))
    return jnp.maximum(fwd, bwd)


def _combine(a1, a2, b1, b2):
    n1 = jnp.maximum(a1, b1)
    n2 = jnp.maximum(jnp.minimum(a1, b1), jnp.maximum(a2, b2))
    return n1, n2


_mesh = plsc.VectorSubcoreMesh(core_axis_name="c", subcore_axis_name="s")


@functools.partial(
    pl.kernel,
    mesh=_mesh,
    out_type=jax.ShapeDtypeStruct((NW, L), jnp.float32),
    scratch_types=[
        pltpu.VMEM((N,), jnp.float32),
        pltpu.VMEM((N,), jnp.float32),
        pltpu.VMEM((L,), jnp.float32),
        pltpu.SemaphoreType.DMA,
        pltpu.SemaphoreType.DMA,
    ],
    compiler_params=pltpu.CompilerParams(needs_layout_passes=False),
)
def _top2_margin(logits_hbm, out_hbm, buf0, buf1, res_v, sem0, sem1):
    cid = lax.axis_index("c")
    sid = lax.axis_index("s")
    wid = cid * 16 + sid
    base = wid * ROWS_PER_W
    sems = (sem0, sem1)
    bufs = (buf0, buf1)

    copies = [None, None]
    copies[0] = pltpu.async_copy(logits_hbm.at[base], bufs[0], sems[0])

    res = jnp.zeros((L,), jnp.float32)
    for j in range(ROWS_PER_W):
        nxt = (j + 1) % 2
        if j + 1 < ROWS_PER_W:
            copies[nxt] = pltpu.async_copy(
                logits_hbm.at[base + j + 1], bufs[nxt], sems[nxt]
            )
        copies[j % 2].wait()
        row = bufs[j % 2]

        neg = jnp.full((L,), -jnp.inf, jnp.float32)
        init = (tuple([neg] * ACC), tuple([neg] * ACC))

        def body(i, carry):
            m1s, m2s = carry
            n1, n2 = [], []
            for a in range(ACC):
                x = row[pl.ds((i * ACC + a) * L, L)]
                n1.append(jnp.maximum(m1s[a], x))
                n2.append(jnp.maximum(m2s[a], jnp.minimum(m1s[a], x)))
            return tuple(n1), tuple(n2)

        m1l, m2l = lax.fori_loop(0, STEPS, body, init)
        m1l, m2l = list(m1l), list(m2l)
        while len(m1l) > 1:
            n1, n2 = [], []
            for a in range(0, len(m1l), 2):
                c1, c2 = _combine(m1l[a], m2l[a], m1l[a + 1], m2l[a + 1])
                n1.append(c1)
                n2.append(c2)
            m1l, m2l = n1, n2
        m1, m2 = m1l[0], m2l[0]

        s1v = _bcast_max(m1)
        maskv = m1 == s1v
        cntv = plsc.all_reduce_population_count(maskv)
        t = jnp.where(maskv, m2, m1)
        s2v = jnp.where(cntv >= 2, s1v, _bcast_max(t))
        margin = s1v - s2v

        lane = lax.iota(jnp.int32, L)
        res = jnp.where(lane == j, margin, res)

    res_v[...] = res
    pltpu.sync_copy(res_v, out_hbm.at[wid])


def kernel(logits):
    out = _top2_margin(logits)
    return out[:, :ROWS_PER_W].reshape(R)
